# initial kernel scaffold (unmeasured)
import jax
import jax.numpy as jnp
from jax import lax
from jax.experimental import pallas as pl
from jax.experimental.pallas import tpu as pltpu

N_DEV = 16


def kernel(x, w_mat):
    m_per, k = x.shape
    n = w_mat.shape[1]
    n_per = n // N_DEV

    def body(x_ref, w_ref, out_ref, comm_ref, send_sems, recv_sems):
        me = lax.axis_index("i")

        y = jnp.dot(x_ref[:, :], w_ref[:, :], preferred_element_type=jnp.float32)
        c = 0.7978845608028654
        y = 0.5 * y * (1.0 + jnp.tanh(c * (y + 0.044715 * y * y * y)))

        for j in range(N_DEV):
            comm_ref[j, :, :] = y[:, j * n_per:(j + 1) * n_per]

        rdmas = []
        for d in range(1, N_DEV):
            tgt = lax.rem(me + d, N_DEV)
            rdma = pltpu.make_async_remote_copy(
                src_ref=comm_ref.at[tgt],
                dst_ref=out_ref.at[pl.ds(me * m_per, m_per), :],
                send_sem=send_sems.at[d],
                recv_sem=recv_sems.at[d],
                device_id=(tgt,),
                device_id_type=pl.DeviceIdType.MESH,
            )
            rdma.start()
            rdmas.append(rdma)

        out_ref[pl.ds(me * m_per, m_per), :] = comm_ref[me, :, :]

        for rdma in rdmas:
            rdma.wait()

    return pl.pallas_call(
        body,
        out_shape=jax.ShapeDtypeStruct((N_DEV * m_per, n_per), jnp.float32),
        in_specs=[
            pl.BlockSpec(memory_space=pltpu.VMEM),
            pl.BlockSpec(memory_space=pltpu.VMEM),
        ],
        out_specs=pl.BlockSpec(memory_space=pltpu.VMEM),
        scratch_shapes=[
            pltpu.VMEM((N_DEV, m_per, n_per), jnp.float32),
            pltpu.SemaphoreType.DMA((N_DEV,)),
            pltpu.SemaphoreType.DMA((N_DEV,)),
        ],
        compiler_params=pltpu.CompilerParams(collective_id=0),
    )(x, w_mat)


# baseline (device time: 19812 ns/iter reference)
import jax
import jax.numpy as jnp
from jax import lax
from jax.experimental import pallas as pl
from jax.experimental.pallas import tpu as pltpu

N_DEV = 16


def kernel(x, w_mat):
    m_per, k = x.shape
    n = w_mat.shape[1]
    n_per = n // N_DEV

    def body(x_ref, w_ref, out_ref, comm_ref, send_sems, recv_sems):
        me = lax.axis_index("i")

        y = jnp.dot(x_ref[:, :], w_ref[:, :], preferred_element_type=jnp.float32)
        c = 0.7978845608028654
        y = 0.5 * y * (1.0 + jnp.tanh(c * (y + 0.044715 * y * y * y)))

        for j in range(N_DEV):
            comm_ref[j, :, :] = y[:, j * n_per:(j + 1) * n_per]

        rdmas = []
        for d in range(1, N_DEV):
            tgt = lax.rem(me + d, N_DEV)
            rdma = pltpu.make_async_remote_copy(
                src_ref=comm_ref.at[tgt],
                dst_ref=out_ref.at[pl.ds(me * m_per, m_per), :],
                send_sem=send_sems.at[d],
                recv_sem=recv_sems.at[d],
                device_id=(tgt,),
                device_id_type=pl.DeviceIdType.MESH,
            )
            rdma.start()
            rdmas.append(rdma)

        out_ref[pl.ds(me * m_per, m_per), :] = comm_ref[me, :, :]

        for rdma in rdmas:
            rdma.wait()

    return pl.pallas_call(
        body,
        out_shape=jax.ShapeDtypeStruct((N_DEV * m_per, n_per), jnp.float32),
        in_specs=[
            pl.BlockSpec(memory_space=pltpu.VMEM),
            pl.BlockSpec(memory_space=pltpu.VMEM),
        ],
        out_specs=pl.BlockSpec(memory_space=pltpu.VMEM),
        scratch_shapes=[
            pltpu.VMEM((N_DEV, m_per, n_per), jnp.float32),
            pltpu.SemaphoreType.DMA((N_DEV,)),
            pltpu.SemaphoreType.DMA((N_DEV,)),
        ],
    )(x, w_mat)


# device time: 15681 ns/iter; 1.2634x vs baseline; 1.2634x over previous
import os

import jax
import jax.numpy as jnp
from jax import lax
from jax.experimental import pallas as pl
from jax.experimental.pallas import tpu as pltpu

N_DEV = 16
_KVAR = os.environ.get("KVAR", "v3")
_WBLK = int(os.environ.get("WBLK", "256"))
_BLOCKER_MB = int(os.environ.get("BLOCKER_MB", "64"))


def kernel(x, w_mat):
    m_per, k = x.shape
    n = w_mat.shape[1]
    n_per = n // N_DEV

    if _KVAR == "now":
        def body_now(x_ref, w_ref, out_ref):
            for j in range(N_DEV):
                out_ref[pl.ds(j * m_per, m_per), :] = x_ref[:, 0:n_per]

        return pl.pallas_call(
            body_now,
            out_shape=jax.ShapeDtypeStruct((N_DEV * m_per, n_per), jnp.float32),
            in_specs=[
                pl.BlockSpec(memory_space=pltpu.VMEM),
                pl.BlockSpec(memory_space=pltpu.MemorySpace.HBM),
            ],
            out_specs=pl.BlockSpec(memory_space=pltpu.VMEM),
        )(x, w_mat)

    if _KVAR in ("any", "anysplit"):
        def body_any(x_ref, w_ref, out_ref, w_vmem, sems):
            if _KVAR == "any":
                cp = pltpu.make_async_copy(w_ref, w_vmem, sems.at[0])
                cp.start()
                cp.wait()
            else:
                kq = k // 4
                cps = []
                for q in range(4):
                    cp = pltpu.make_async_copy(
                        w_ref.at[pl.ds(q * kq, kq), :],
                        w_vmem.at[pl.ds(q * kq, kq), :],
                        sems.at[q],
                    )
                    cp.start()
                    cps.append(cp)
                for cp in cps:
                    cp.wait()
            out_ref[pl.ds(0, m_per), :] = w_vmem[0:m_per, 0:n_per] + x_ref[:, 0:n_per]

        return pl.pallas_call(
            body_any,
            out_shape=jax.ShapeDtypeStruct((N_DEV * m_per, n_per), jnp.float32),
            in_specs=[
                pl.BlockSpec(memory_space=pltpu.VMEM),
                pl.BlockSpec(memory_space=pl.ANY),
            ],
            out_specs=pl.BlockSpec(memory_space=pltpu.VMEM),
            scratch_shapes=[
                pltpu.VMEM((k, n), jnp.float32),
                pltpu.SemaphoreType.DMA((4,)),
            ],
        )(x, w_mat)

    if _KVAR == "wdma":
        def body_wdma(x_ref, w_ref, out_ref, w_vmem, sem):
            cp = pltpu.make_async_copy(w_ref, w_vmem, sem)
            cp.start()
            cp.wait()
            out_ref[pl.ds(0, m_per), :] = w_vmem[0:m_per, 0:n_per] + x_ref[:, 0:n_per]

        return pl.pallas_call(
            body_wdma,
            out_shape=jax.ShapeDtypeStruct((N_DEV * m_per, n_per), jnp.float32),
            in_specs=[
                pl.BlockSpec(memory_space=pltpu.VMEM),
                pl.BlockSpec(memory_space=pltpu.MemorySpace.HBM),
            ],
            out_specs=pl.BlockSpec(memory_space=pltpu.VMEM),
            scratch_shapes=[
                pltpu.VMEM((k, n), jnp.float32),
                pltpu.SemaphoreType.DMA,
            ],
        )(x, w_mat)

    if _KVAR == "v3":
        nblk = n // _WBLK
        cpb = _WBLK // n_per
        c_gelu = 0.7978845608028654

        def body_v3(x_hbm, w_hbm, out_ref, xb, wb, comm_ref,
                    blocker, wsems, xsem, send_sems, recv_sems):
            me = lax.axis_index("i")

            barrier_sem = pltpu.get_barrier_semaphore()
            for dd in range(1, N_DEV):
                pl.semaphore_signal(
                    barrier_sem, inc=1,
                    device_id=(lax.rem(me + dd, N_DEV),),
                    device_id_type=pl.DeviceIdType.MESH,
                )

            xcp = pltpu.make_async_copy(x_hbm, xb, xsem)
            xcp.start()

            def blk_idx(b):
                return lax.rem(me // cpb + 1 + b, nblk)

            wcps = {}

            def start_wdma(b):
                bi = blk_idx(b)
                cp = pltpu.make_async_copy(
                    w_hbm.at[:, pl.ds(bi * _WBLK, _WBLK)],
                    wb.at[b % 2],
                    wsems.at[b % 2],
                )
                cp.start()
                wcps[b] = cp

            start_wdma(0)
            xcp.wait()
            start_wdma(1)

            rdmas = []
            for b in range(nblk):
                wcps[b].wait()
                y = jnp.dot(xb[:, :], wb[b % 2], preferred_element_type=jnp.float32)
                if b + 2 < nblk:
                    start_wdma(b + 2)
                y = 0.5 * y * (1.0 + jnp.tanh(c_gelu * (y + 0.044715 * y * y * y)))
                if b == 0:
                    pl.semaphore_wait(barrier_sem, N_DEV - 1)
                bi = blk_idx(b)
                for t in range(cpb):
                    tgt = bi * cpb + t
                    comm_ref[tgt, :, :] = y[:, t * n_per:(t + 1) * n_per]
                    rdma = pltpu.make_async_remote_copy(
                        src_ref=comm_ref.at[tgt],
                        dst_ref=out_ref.at[pl.ds(me * m_per, m_per), :],
                        send_sem=send_sems.at[tgt],
                        recv_sem=recv_sems.at[me],
                        device_id=(tgt,),
                        device_id_type=pl.DeviceIdType.MESH,
                    )

                    @pl.when(tgt != me)
                    def _(rdma=rdma):
                        rdma.start()

                    @pl.when(tgt == me)
                    def _(tgt=tgt):
                        out_ref[pl.ds(me * m_per, m_per), :] = comm_ref[tgt, :, :]

                    rdmas.append((tgt, rdma))

            for tgt, rdma in rdmas:
                @pl.when(tgt != me)
                def _(rdma=rdma):
                    rdma.wait_send()
            for dd in range(1, N_DEV):
                src = lax.rem(me + dd, N_DEV)
                recv = pltpu.make_async_remote_copy(
                    src_ref=comm_ref.at[src],
                    dst_ref=out_ref.at[pl.ds(src * m_per, m_per), :],
                    send_sem=send_sems.at[src],
                    recv_sem=recv_sems.at[src],
                    device_id=(src,),
                    device_id_type=pl.DeviceIdType.MESH,
                )
                recv.wait_recv()

        return pl.pallas_call(
            body_v3,
            out_shape=jax.ShapeDtypeStruct((N_DEV * m_per, n_per), jnp.float32),
            in_specs=[
                pl.BlockSpec(memory_space=pl.ANY),
                pl.BlockSpec(memory_space=pl.ANY),
            ],
            out_specs=pl.BlockSpec(memory_space=pltpu.VMEM),
            scratch_shapes=[
                pltpu.VMEM((m_per, k), jnp.float32),
                pltpu.VMEM((2, k, _WBLK), jnp.float32),
                pltpu.VMEM((N_DEV, m_per, n_per), jnp.float32),
                pltpu.VMEM((_BLOCKER_MB * 1024 * 256,), jnp.float32),
                pltpu.SemaphoreType.DMA((2,)),
                pltpu.SemaphoreType.DMA,
                pltpu.SemaphoreType.DMA((N_DEV,)),
                pltpu.SemaphoreType.DMA((N_DEV,)),
            ],
            compiler_params=pltpu.CompilerParams(collective_id=0),
        )(x, w_mat)

    if _KVAR == "r1c":
        def body_r1c(x_ref, w_ref, out_ref, comm_ref, recv_buf, send_sems, recv_sems):
            me = lax.axis_index("i")
            barrier_sem = pltpu.get_barrier_semaphore()
            for d in range(1, N_DEV):
                tgt = lax.rem(me + d, N_DEV)
                pl.semaphore_signal(
                    barrier_sem, inc=1,
                    device_id=(tgt,), device_id_type=pl.DeviceIdType.MESH,
                )
            pl.semaphore_wait(barrier_sem, N_DEV - 1)

            tgt = lax.rem(me + 1, N_DEV)
            rdma = pltpu.make_async_remote_copy(
                src_ref=comm_ref.at[tgt],
                dst_ref=recv_buf.at[1],
                send_sem=send_sems.at[1],
                recv_sem=recv_sems.at[1],
                device_id=(tgt,),
                device_id_type=pl.DeviceIdType.MESH,
            )
            rdma.start()

            y = jnp.dot(x_ref[:, :], w_ref[:, :], preferred_element_type=jnp.float32)
            c = 0.7978845608028654
            y = 0.5 * y * (1.0 + jnp.tanh(c * (y + 0.044715 * y * y * y)))
            for j in range(N_DEV):
                out_ref[pl.ds(j * m_per, m_per), :] = y[:, j * n_per:(j + 1) * n_per]

            rdma.wait()

        return pl.pallas_call(
            body_r1c,
            out_shape=jax.ShapeDtypeStruct((N_DEV * m_per, n_per), jnp.float32),
            in_specs=[
                pl.BlockSpec(memory_space=pltpu.VMEM),
                pl.BlockSpec(memory_space=pltpu.VMEM),
            ],
            out_specs=pl.BlockSpec(memory_space=pltpu.VMEM),
            scratch_shapes=[
                pltpu.VMEM((N_DEV, m_per, n_per), jnp.float32),
                pltpu.VMEM((N_DEV, m_per, n_per), jnp.float32),
                pltpu.SemaphoreType.DMA((N_DEV,)),
                pltpu.SemaphoreType.DMA((N_DEV,)),
            ],
            compiler_params=pltpu.CompilerParams(collective_id=0),
        )(x, w_mat)

    def body(x_ref, w_ref, out_ref, comm_ref, recv_buf, send_sems, recv_sems):
        me = lax.axis_index("i")

        if _KVAR == "wcopy":
            out_ref[pl.ds(0, m_per), :] = w_ref[0:m_per, 0:n_per] + x_ref[:, 0:n_per]
            return

        if _KVAR == "bf16":
            y = jnp.dot(
                x_ref[:, :].astype(jnp.bfloat16),
                w_ref[:, :].astype(jnp.bfloat16),
                preferred_element_type=jnp.float32,
            )
        elif _KVAR == "hi":
            y = jnp.dot(
                x_ref[:, :], w_ref[:, :],
                preferred_element_type=jnp.float32,
                precision=lax.Precision.HIGHEST,
            )
        else:
            y = jnp.dot(x_ref[:, :], w_ref[:, :], preferred_element_type=jnp.float32)
        if _KVAR != "gemm":
            c = 0.7978845608028654
            y = 0.5 * y * (1.0 + jnp.tanh(c * (y + 0.044715 * y * y * y)))

        if _KVAR in ("gemm", "gemmgelu", "bf16", "hi"):
            out_ref[pl.ds(0, m_per), :] = y[:, 0:n_per]
            return

        for j in range(N_DEV):
            comm_ref[j, :, :] = y[:, j * n_per:(j + 1) * n_per]

        if _KVAR == "compute":
            for j in range(N_DEV):
                out_ref[pl.ds(j * m_per, m_per), :] = comm_ref[j, :, :]
            return

        if _KVAR in ("bar", "v2", "r1", "r4", "r1s", "v2s", "r1t", "r1l"):
            barrier_sem = pltpu.get_barrier_semaphore()
            for d in range(1, N_DEV):
                tgt = lax.rem(me + d, N_DEV)
                pl.semaphore_signal(
                    barrier_sem, inc=1,
                    device_id=(tgt,), device_id_type=pl.DeviceIdType.MESH,
                )
            pl.semaphore_wait(barrier_sem, N_DEV - 1)

        n_rdma = {"bar": 0, "r1": 1, "r1s": 1, "r1t": 1, "r1l": 1, "r4": 4}.get(
            _KVAR, N_DEV - 1)
        to_scratch = _KVAR in ("r1s", "v2s", "r1t", "r1l")
        rdmas = []
        for d in range(1, 1 + n_rdma):
            tgt = lax.rem(me + d, N_DEV)
            if _KVAR == "r1t":
                rdma = pltpu.make_async_remote_copy(
                    src_ref=comm_ref.at[tgt, pl.ds(0, 8), :],
                    dst_ref=recv_buf.at[d, pl.ds(0, 8), :],
                    send_sem=send_sems.at[d],
                    recv_sem=recv_sems.at[d],
                    device_id=(tgt,),
                    device_id_type=pl.DeviceIdType.MESH,
                )
            elif _KVAR == "r1l":
                rdma = pltpu.make_async_remote_copy(
                    src_ref=comm_ref.at[tgt],
                    dst_ref=recv_buf.at[d],
                    send_sem=send_sems.at[d],
                    recv_sem=recv_sems.at[d],
                    device_id=tgt,
                    device_id_type=pl.DeviceIdType.LOGICAL,
                )
            else:
                rdma = pltpu.make_async_remote_copy(
                    src_ref=comm_ref.at[tgt],
                    dst_ref=recv_buf.at[d] if to_scratch
                    else out_ref.at[pl.ds(me * m_per, m_per), :],
                    send_sem=send_sems.at[d],
                    recv_sem=recv_sems.at[d],
                    device_id=(tgt,),
                    device_id_type=pl.DeviceIdType.MESH,
                )
            rdma.start()
            rdmas.append(rdma)

        out_ref[pl.ds(me * m_per, m_per), :] = comm_ref[me, :, :]
        if _KVAR in ("bar", "r1", "r4", "r1s", "r1t", "r1l"):
            for j in range(N_DEV):
                out_ref[pl.ds(j * m_per, m_per), :] = comm_ref[j, :, :]

        for rdma in rdmas:
            rdma.wait()

        if to_scratch and _KVAR == "v2s":
            for d in range(1, N_DEV):
                src = lax.rem(me - d + N_DEV, N_DEV)
                out_ref[pl.ds(src * m_per, m_per), :] = recv_buf[d, :, :]

    params = {}
    if _KVAR in ("bar", "v2", "r1", "r4", "r1s", "v2s", "r1t", "r1l"):
        params["compiler_params"] = pltpu.CompilerParams(collective_id=0)
    return pl.pallas_call(
        body,
        out_shape=jax.ShapeDtypeStruct((N_DEV * m_per, n_per), jnp.float32),
        in_specs=[
            pl.BlockSpec(memory_space=pltpu.VMEM),
            pl.BlockSpec(memory_space=pltpu.VMEM),
        ],
        out_specs=pl.BlockSpec(memory_space=pltpu.VMEM),
        scratch_shapes=[
            pltpu.VMEM((N_DEV, m_per, n_per), jnp.float32),
            pltpu.VMEM((N_DEV, m_per, n_per), jnp.float32),
            pltpu.SemaphoreType.DMA((N_DEV,)),
            pltpu.SemaphoreType.DMA((N_DEV,)),
        ],
        **params,
    )(x, w_mat)


# device time: 14430 ns/iter; 1.3730x vs baseline; 1.0867x over previous
import os

import jax
import jax.numpy as jnp
from jax import lax
from jax.experimental import pallas as pl
from jax.experimental.pallas import tpu as pltpu

N_DEV = 16
_KVAR = os.environ.get("KVAR", "v3")
_WBLK = int(os.environ.get("WBLK", "256"))
_BLOCKER_MB = int(os.environ.get("BLOCKER_MB", "56"))


def kernel(x, w_mat):
    m_per, k = x.shape
    n = w_mat.shape[1]
    n_per = n // N_DEV

    if _KVAR == "now":
        def body_now(x_ref, w_ref, out_ref):
            for j in range(N_DEV):
                out_ref[pl.ds(j * m_per, m_per), :] = x_ref[:, 0:n_per]

        return pl.pallas_call(
            body_now,
            out_shape=jax.ShapeDtypeStruct((N_DEV * m_per, n_per), jnp.float32),
            in_specs=[
                pl.BlockSpec(memory_space=pltpu.VMEM),
                pl.BlockSpec(memory_space=pltpu.MemorySpace.HBM),
            ],
            out_specs=pl.BlockSpec(memory_space=pltpu.VMEM),
        )(x, w_mat)

    if _KVAR in ("any", "anysplit"):
        def body_any(x_ref, w_ref, out_ref, w_vmem, sems):
            if _KVAR == "any":
                cp = pltpu.make_async_copy(w_ref, w_vmem, sems.at[0])
                cp.start()
                cp.wait()
            else:
                kq = k // 4
                cps = []
                for q in range(4):
                    cp = pltpu.make_async_copy(
                        w_ref.at[pl.ds(q * kq, kq), :],
                        w_vmem.at[pl.ds(q * kq, kq), :],
                        sems.at[q],
                    )
                    cp.start()
                    cps.append(cp)
                for cp in cps:
                    cp.wait()
            out_ref[pl.ds(0, m_per), :] = w_vmem[0:m_per, 0:n_per] + x_ref[:, 0:n_per]

        return pl.pallas_call(
            body_any,
            out_shape=jax.ShapeDtypeStruct((N_DEV * m_per, n_per), jnp.float32),
            in_specs=[
                pl.BlockSpec(memory_space=pltpu.VMEM),
                pl.BlockSpec(memory_space=pl.ANY),
            ],
            out_specs=pl.BlockSpec(memory_space=pltpu.VMEM),
            scratch_shapes=[
                pltpu.VMEM((k, n), jnp.float32),
                pltpu.SemaphoreType.DMA((4,)),
            ],
        )(x, w_mat)

    if _KVAR == "wdma":
        def body_wdma(x_ref, w_ref, out_ref, w_vmem, sem):
            cp = pltpu.make_async_copy(w_ref, w_vmem, sem)
            cp.start()
            cp.wait()
            out_ref[pl.ds(0, m_per), :] = w_vmem[0:m_per, 0:n_per] + x_ref[:, 0:n_per]

        return pl.pallas_call(
            body_wdma,
            out_shape=jax.ShapeDtypeStruct((N_DEV * m_per, n_per), jnp.float32),
            in_specs=[
                pl.BlockSpec(memory_space=pltpu.VMEM),
                pl.BlockSpec(memory_space=pltpu.MemorySpace.HBM),
            ],
            out_specs=pl.BlockSpec(memory_space=pltpu.VMEM),
            scratch_shapes=[
                pltpu.VMEM((k, n), jnp.float32),
                pltpu.SemaphoreType.DMA,
            ],
        )(x, w_mat)

    if os.environ.get("OB") == "1":
        x, w_mat = lax.optimization_barrier((x, w_mat))

    if _KVAR == "v3":
        nblk = n // _WBLK
        cpb = _WBLK // n_per
        c_gelu = 0.7978845608028654

        def body_v3(x_hbm, w_hbm, out_ref, xb, wb, comm_ref,
                    blocker, wsems, xsem, send_sems, recv_sems):
            me = lax.axis_index("i")

            barrier_sem = pltpu.get_barrier_semaphore()
            for dd in range(1, N_DEV):
                pl.semaphore_signal(
                    barrier_sem, inc=1,
                    device_id=(lax.rem(me + dd, N_DEV),),
                    device_id_type=pl.DeviceIdType.MESH,
                )

            xcp = pltpu.make_async_copy(x_hbm, xb, xsem)
            xcp.start()

            def blk_idx(b):
                return lax.rem(me // cpb + 1 + b, nblk)

            wcps = {}
            for b in range(nblk):
                bi = blk_idx(b)
                cp = pltpu.make_async_copy(
                    w_hbm.at[:, pl.ds(bi * _WBLK, _WBLK)],
                    wb.at[b],
                    wsems.at[b],
                )
                cp.start()
                wcps[b] = cp
            xcp.wait()

            rdmas = []
            for b in range(nblk):
                wcps[b].wait()
                y = jnp.dot(xb[:, :], wb[b], preferred_element_type=jnp.float32)
                y = 0.5 * y * (1.0 + jnp.tanh(c_gelu * (y + 0.044715 * y * y * y)))
                if b == 0:
                    pl.semaphore_wait(barrier_sem, N_DEV - 1)
                bi = blk_idx(b)
                for t in range(cpb):
                    tgt = bi * cpb + t
                    comm_ref[tgt, :, :] = y[:, t * n_per:(t + 1) * n_per]
                    rdma = pltpu.make_async_remote_copy(
                        src_ref=comm_ref.at[tgt],
                        dst_ref=out_ref.at[pl.ds(me * m_per, m_per), :],
                        send_sem=send_sems.at[tgt],
                        recv_sem=recv_sems.at[me],
                        device_id=(tgt,),
                        device_id_type=pl.DeviceIdType.MESH,
                    )

                    @pl.when(tgt != me)
                    def _(rdma=rdma):
                        rdma.start()

                    @pl.when(tgt == me)
                    def _(tgt=tgt):
                        out_ref[pl.ds(me * m_per, m_per), :] = comm_ref[tgt, :, :]

                    rdmas.append((tgt, rdma))

            for tgt, rdma in rdmas:
                @pl.when(tgt != me)
                def _(rdma=rdma):
                    rdma.wait_send()
            for dd in range(1, N_DEV):
                src = lax.rem(me + dd, N_DEV)
                recv = pltpu.make_async_remote_copy(
                    src_ref=comm_ref.at[src],
                    dst_ref=out_ref.at[pl.ds(src * m_per, m_per), :],
                    send_sem=send_sems.at[src],
                    recv_sem=recv_sems.at[src],
                    device_id=(src,),
                    device_id_type=pl.DeviceIdType.MESH,
                )
                recv.wait_recv()

        return pl.pallas_call(
            body_v3,
            out_shape=jax.ShapeDtypeStruct((N_DEV * m_per, n_per), jnp.float32),
            in_specs=[
                pl.BlockSpec(memory_space=pl.ANY),
                pl.BlockSpec(memory_space=pl.ANY),
            ],
            out_specs=pl.BlockSpec(memory_space=pltpu.VMEM),
            scratch_shapes=[
                pltpu.VMEM((m_per, k), jnp.float32),
                pltpu.VMEM((nblk, k, _WBLK), jnp.float32),
                pltpu.VMEM((N_DEV, m_per, n_per), jnp.float32),
                pltpu.VMEM((_BLOCKER_MB * 1024 * 256,), jnp.float32),
                pltpu.SemaphoreType.DMA((nblk,)),
                pltpu.SemaphoreType.DMA,
                pltpu.SemaphoreType.DMA((N_DEV,)),
                pltpu.SemaphoreType.DMA((N_DEV,)),
            ],
            compiler_params=pltpu.CompilerParams(
                collective_id=0,
                vmem_limit_bytes=int(os.environ.get("VLIM_MB", "64")) * 1024 * 1024,
            ),
        )(x, w_mat)

    if _KVAR == "r1c":
        def body_r1c(x_ref, w_ref, out_ref, comm_ref, recv_buf, send_sems, recv_sems):
            me = lax.axis_index("i")
            barrier_sem = pltpu.get_barrier_semaphore()
            for d in range(1, N_DEV):
                tgt = lax.rem(me + d, N_DEV)
                pl.semaphore_signal(
                    barrier_sem, inc=1,
                    device_id=(tgt,), device_id_type=pl.DeviceIdType.MESH,
                )
            pl.semaphore_wait(barrier_sem, N_DEV - 1)

            tgt = lax.rem(me + 1, N_DEV)
            rdma = pltpu.make_async_remote_copy(
                src_ref=comm_ref.at[tgt],
                dst_ref=recv_buf.at[1],
                send_sem=send_sems.at[1],
                recv_sem=recv_sems.at[1],
                device_id=(tgt,),
                device_id_type=pl.DeviceIdType.MESH,
            )
            rdma.start()

            y = jnp.dot(x_ref[:, :], w_ref[:, :], preferred_element_type=jnp.float32)
            c = 0.7978845608028654
            y = 0.5 * y * (1.0 + jnp.tanh(c * (y + 0.044715 * y * y * y)))
            for j in range(N_DEV):
                out_ref[pl.ds(j * m_per, m_per), :] = y[:, j * n_per:(j + 1) * n_per]

            rdma.wait()

        return pl.pallas_call(
            body_r1c,
            out_shape=jax.ShapeDtypeStruct((N_DEV * m_per, n_per), jnp.float32),
            in_specs=[
                pl.BlockSpec(memory_space=pltpu.VMEM),
                pl.BlockSpec(memory_space=pltpu.VMEM),
            ],
            out_specs=pl.BlockSpec(memory_space=pltpu.VMEM),
            scratch_shapes=[
                pltpu.VMEM((N_DEV, m_per, n_per), jnp.float32),
                pltpu.VMEM((N_DEV, m_per, n_per), jnp.float32),
                pltpu.SemaphoreType.DMA((N_DEV,)),
                pltpu.SemaphoreType.DMA((N_DEV,)),
            ],
            compiler_params=pltpu.CompilerParams(collective_id=0),
        )(x, w_mat)

    def body(x_ref, w_ref, out_ref, comm_ref, recv_buf, send_sems, recv_sems):
        me = lax.axis_index("i")

        if _KVAR == "wcopy":
            out_ref[pl.ds(0, m_per), :] = w_ref[0:m_per, 0:n_per] + x_ref[:, 0:n_per]
            return

        if _KVAR == "bf16":
            y = jnp.dot(
                x_ref[:, :].astype(jnp.bfloat16),
                w_ref[:, :].astype(jnp.bfloat16),
                preferred_element_type=jnp.float32,
            )
        elif _KVAR == "hi":
            y = jnp.dot(
                x_ref[:, :], w_ref[:, :],
                preferred_element_type=jnp.float32,
                precision=lax.Precision.HIGHEST,
            )
        else:
            y = jnp.dot(x_ref[:, :], w_ref[:, :], preferred_element_type=jnp.float32)
        if _KVAR != "gemm":
            c = 0.7978845608028654
            y = 0.5 * y * (1.0 + jnp.tanh(c * (y + 0.044715 * y * y * y)))

        if _KVAR in ("gemm", "gemmgelu", "bf16", "hi"):
            out_ref[pl.ds(0, m_per), :] = y[:, 0:n_per]
            return

        for j in range(N_DEV):
            comm_ref[j, :, :] = y[:, j * n_per:(j + 1) * n_per]

        if _KVAR == "compute":
            for j in range(N_DEV):
                out_ref[pl.ds(j * m_per, m_per), :] = comm_ref[j, :, :]
            return

        if _KVAR in ("bar", "v2", "r1", "r4", "r1s", "v2s", "r1t", "r1l"):
            barrier_sem = pltpu.get_barrier_semaphore()
            for d in range(1, N_DEV):
                tgt = lax.rem(me + d, N_DEV)
                pl.semaphore_signal(
                    barrier_sem, inc=1,
                    device_id=(tgt,), device_id_type=pl.DeviceIdType.MESH,
                )
            pl.semaphore_wait(barrier_sem, N_DEV - 1)

        n_rdma = {"bar": 0, "r1": 1, "r1s": 1, "r1t": 1, "r1l": 1, "r4": 4}.get(
            _KVAR, N_DEV - 1)
        to_scratch = _KVAR in ("r1s", "v2s", "r1t", "r1l")
        rdmas = []
        for d in range(1, 1 + n_rdma):
            tgt = lax.rem(me + d, N_DEV)
            if _KVAR == "r1t":
                rdma = pltpu.make_async_remote_copy(
                    src_ref=comm_ref.at[tgt, pl.ds(0, 8), :],
                    dst_ref=recv_buf.at[d, pl.ds(0, 8), :],
                    send_sem=send_sems.at[d],
                    recv_sem=recv_sems.at[d],
                    device_id=(tgt,),
                    device_id_type=pl.DeviceIdType.MESH,
                )
            elif _KVAR == "r1l":
                rdma = pltpu.make_async_remote_copy(
                    src_ref=comm_ref.at[tgt],
                    dst_ref=recv_buf.at[d],
                    send_sem=send_sems.at[d],
                    recv_sem=recv_sems.at[d],
                    device_id=tgt,
                    device_id_type=pl.DeviceIdType.LOGICAL,
                )
            else:
                rdma = pltpu.make_async_remote_copy(
                    src_ref=comm_ref.at[tgt],
                    dst_ref=recv_buf.at[d] if to_scratch
                    else out_ref.at[pl.ds(me * m_per, m_per), :],
                    send_sem=send_sems.at[d],
                    recv_sem=recv_sems.at[d],
                    device_id=(tgt,),
                    device_id_type=pl.DeviceIdType.MESH,
                )
            rdma.start()
            rdmas.append(rdma)

        out_ref[pl.ds(me * m_per, m_per), :] = comm_ref[me, :, :]
        if _KVAR in ("bar", "r1", "r4", "r1s", "r1t", "r1l"):
            for j in range(N_DEV):
                out_ref[pl.ds(j * m_per, m_per), :] = comm_ref[j, :, :]

        for rdma in rdmas:
            rdma.wait()

        if to_scratch and _KVAR == "v2s":
            for d in range(1, N_DEV):
                src = lax.rem(me - d + N_DEV, N_DEV)
                out_ref[pl.ds(src * m_per, m_per), :] = recv_buf[d, :, :]

    params = {}
    if _KVAR in ("bar", "v2", "r1", "r4", "r1s", "v2s", "r1t", "r1l"):
        params["compiler_params"] = pltpu.CompilerParams(collective_id=0)
    return pl.pallas_call(
        body,
        out_shape=jax.ShapeDtypeStruct((N_DEV * m_per, n_per), jnp.float32),
        in_specs=[
            pl.BlockSpec(memory_space=pltpu.VMEM),
            pl.BlockSpec(memory_space=pltpu.VMEM),
        ],
        out_specs=pl.BlockSpec(memory_space=pltpu.VMEM),
        scratch_shapes=[
            pltpu.VMEM((N_DEV, m_per, n_per), jnp.float32),
            pltpu.VMEM((N_DEV, m_per, n_per), jnp.float32),
            pltpu.SemaphoreType.DMA((N_DEV,)),
            pltpu.SemaphoreType.DMA((N_DEV,)),
        ],
        **params,
    )(x, w_mat)


# device time: 14320 ns/iter; 1.3835x vs baseline; 1.0077x over previous
import os

import jax
import jax.numpy as jnp
from jax import lax
from jax.experimental import pallas as pl
from jax.experimental.pallas import tpu as pltpu

N_DEV = 16
_KVAR = os.environ.get("KVAR", "v3")
_WBLK = int(os.environ.get("WBLK", "512"))
_BLOCKER_MB = int(os.environ.get("BLOCKER_MB", "56"))


def kernel(x, w_mat):
    m_per, k = x.shape
    n = w_mat.shape[1]
    n_per = n // N_DEV

    if _KVAR == "now":
        def body_now(x_ref, w_ref, out_ref):
            for j in range(N_DEV):
                out_ref[pl.ds(j * m_per, m_per), :] = x_ref[:, 0:n_per]

        return pl.pallas_call(
            body_now,
            out_shape=jax.ShapeDtypeStruct((N_DEV * m_per, n_per), jnp.float32),
            in_specs=[
                pl.BlockSpec(memory_space=pltpu.VMEM),
                pl.BlockSpec(memory_space=pltpu.MemorySpace.HBM),
            ],
            out_specs=pl.BlockSpec(memory_space=pltpu.VMEM),
        )(x, w_mat)

    if _KVAR in ("any", "anysplit"):
        def body_any(x_ref, w_ref, out_ref, w_vmem, sems):
            if _KVAR == "any":
                cp = pltpu.make_async_copy(w_ref, w_vmem, sems.at[0])
                cp.start()
                cp.wait()
            else:
                kq = k // 4
                cps = []
                for q in range(4):
                    cp = pltpu.make_async_copy(
                        w_ref.at[pl.ds(q * kq, kq), :],
                        w_vmem.at[pl.ds(q * kq, kq), :],
                        sems.at[q],
                    )
                    cp.start()
                    cps.append(cp)
                for cp in cps:
                    cp.wait()
            out_ref[pl.ds(0, m_per), :] = w_vmem[0:m_per, 0:n_per] + x_ref[:, 0:n_per]

        return pl.pallas_call(
            body_any,
            out_shape=jax.ShapeDtypeStruct((N_DEV * m_per, n_per), jnp.float32),
            in_specs=[
                pl.BlockSpec(memory_space=pltpu.VMEM),
                pl.BlockSpec(memory_space=pl.ANY),
            ],
            out_specs=pl.BlockSpec(memory_space=pltpu.VMEM),
            scratch_shapes=[
                pltpu.VMEM((k, n), jnp.float32),
                pltpu.SemaphoreType.DMA((4,)),
            ],
        )(x, w_mat)

    if _KVAR == "wdma":
        def body_wdma(x_ref, w_ref, out_ref, w_vmem, sem):
            cp = pltpu.make_async_copy(w_ref, w_vmem, sem)
            cp.start()
            cp.wait()
            out_ref[pl.ds(0, m_per), :] = w_vmem[0:m_per, 0:n_per] + x_ref[:, 0:n_per]

        return pl.pallas_call(
            body_wdma,
            out_shape=jax.ShapeDtypeStruct((N_DEV * m_per, n_per), jnp.float32),
            in_specs=[
                pl.BlockSpec(memory_space=pltpu.VMEM),
                pl.BlockSpec(memory_space=pltpu.MemorySpace.HBM),
            ],
            out_specs=pl.BlockSpec(memory_space=pltpu.VMEM),
            scratch_shapes=[
                pltpu.VMEM((k, n), jnp.float32),
                pltpu.SemaphoreType.DMA,
            ],
        )(x, w_mat)

    if os.environ.get("OB") == "1":
        x, w_mat = lax.optimization_barrier((x, w_mat))

    if _KVAR == "v3":
        nblk = n // _WBLK
        cpb = _WBLK // n_per
        c_gelu = 0.7978845608028654

        def body_v3(x_hbm, w_hbm, out_ref, xb, wb, comm_ref,
                    blocker, wsems, xsem, send_sems, recv_sems):
            me = lax.axis_index("i")

            barrier_sem = pltpu.get_barrier_semaphore()
            for dd in range(1, N_DEV):
                pl.semaphore_signal(
                    barrier_sem, inc=1,
                    device_id=(lax.rem(me + dd, N_DEV),),
                    device_id_type=pl.DeviceIdType.MESH,
                )

            xcp = pltpu.make_async_copy(x_hbm, xb, xsem)
            xcp.start()

            def blk_idx(b):
                return lax.rem(me // cpb + 1 + b, nblk)

            wcps = {}
            for b in range(nblk):
                bi = blk_idx(b)
                cp = pltpu.make_async_copy(
                    w_hbm.at[:, pl.ds(bi * _WBLK, _WBLK)],
                    wb.at[b],
                    wsems.at[b],
                )
                cp.start()
                wcps[b] = cp
            xcp.wait()

            rdmas = []
            for b in range(nblk):
                wcps[b].wait()
                y = jnp.dot(xb[:, :], wb[b], preferred_element_type=jnp.float32)
                y = 0.5 * y * (1.0 + jnp.tanh(c_gelu * (y + 0.044715 * y * y * y)))
                if b == 0:
                    pl.semaphore_wait(barrier_sem, N_DEV - 1)
                bi = blk_idx(b)
                for t in range(cpb):
                    tgt = bi * cpb + t
                    comm_ref[tgt, :, :] = y[:, t * n_per:(t + 1) * n_per]
                    rdma = pltpu.make_async_remote_copy(
                        src_ref=comm_ref.at[tgt],
                        dst_ref=out_ref.at[pl.ds(me * m_per, m_per), :],
                        send_sem=send_sems.at[tgt],
                        recv_sem=recv_sems.at[me],
                        device_id=(tgt,),
                        device_id_type=pl.DeviceIdType.MESH,
                    )

                    @pl.when(tgt != me)
                    def _(rdma=rdma):
                        rdma.start()

                    @pl.when(tgt == me)
                    def _(tgt=tgt):
                        out_ref[pl.ds(me * m_per, m_per), :] = comm_ref[tgt, :, :]

                    rdmas.append((tgt, rdma))

            for tgt, rdma in rdmas:
                @pl.when(tgt != me)
                def _(rdma=rdma):
                    rdma.wait_send()
            for dd in range(1, N_DEV):
                src = lax.rem(me + dd, N_DEV)
                recv = pltpu.make_async_remote_copy(
                    src_ref=comm_ref.at[src],
                    dst_ref=out_ref.at[pl.ds(src * m_per, m_per), :],
                    send_sem=send_sems.at[src],
                    recv_sem=recv_sems.at[src],
                    device_id=(src,),
                    device_id_type=pl.DeviceIdType.MESH,
                )
                recv.wait_recv()

        return pl.pallas_call(
            body_v3,
            out_shape=jax.ShapeDtypeStruct((N_DEV * m_per, n_per), jnp.float32),
            in_specs=[
                pl.BlockSpec(memory_space=pl.ANY),
                pl.BlockSpec(memory_space=pl.ANY),
            ],
            out_specs=pl.BlockSpec(memory_space=pltpu.VMEM),
            scratch_shapes=[
                pltpu.VMEM((m_per, k), jnp.float32),
                pltpu.VMEM((nblk, k, _WBLK), jnp.float32),
                pltpu.VMEM((N_DEV, m_per, n_per), jnp.float32),
                pltpu.VMEM((_BLOCKER_MB * 1024 * 256,), jnp.float32),
                pltpu.SemaphoreType.DMA((nblk,)),
                pltpu.SemaphoreType.DMA,
                pltpu.SemaphoreType.DMA((N_DEV,)),
                pltpu.SemaphoreType.DMA((N_DEV,)),
            ],
            compiler_params=pltpu.CompilerParams(
                collective_id=0,
                vmem_limit_bytes=int(os.environ.get("VLIM_MB", "64")) * 1024 * 1024,
            ),
        )(x, w_mat)

    if _KVAR == "r1c":
        def body_r1c(x_ref, w_ref, out_ref, comm_ref, recv_buf, send_sems, recv_sems):
            me = lax.axis_index("i")
            barrier_sem = pltpu.get_barrier_semaphore()
            for d in range(1, N_DEV):
                tgt = lax.rem(me + d, N_DEV)
                pl.semaphore_signal(
                    barrier_sem, inc=1,
                    device_id=(tgt,), device_id_type=pl.DeviceIdType.MESH,
                )
            pl.semaphore_wait(barrier_sem, N_DEV - 1)

            tgt = lax.rem(me + 1, N_DEV)
            rdma = pltpu.make_async_remote_copy(
                src_ref=comm_ref.at[tgt],
                dst_ref=recv_buf.at[1],
                send_sem=send_sems.at[1],
                recv_sem=recv_sems.at[1],
                device_id=(tgt,),
                device_id_type=pl.DeviceIdType.MESH,
            )
            rdma.start()

            y = jnp.dot(x_ref[:, :], w_ref[:, :], preferred_element_type=jnp.float32)
            c = 0.7978845608028654
            y = 0.5 * y * (1.0 + jnp.tanh(c * (y + 0.044715 * y * y * y)))
            for j in range(N_DEV):
                out_ref[pl.ds(j * m_per, m_per), :] = y[:, j * n_per:(j + 1) * n_per]

            rdma.wait()

        return pl.pallas_call(
            body_r1c,
            out_shape=jax.ShapeDtypeStruct((N_DEV * m_per, n_per), jnp.float32),
            in_specs=[
                pl.BlockSpec(memory_space=pltpu.VMEM),
                pl.BlockSpec(memory_space=pltpu.VMEM),
            ],
            out_specs=pl.BlockSpec(memory_space=pltpu.VMEM),
            scratch_shapes=[
                pltpu.VMEM((N_DEV, m_per, n_per), jnp.float32),
                pltpu.VMEM((N_DEV, m_per, n_per), jnp.float32),
                pltpu.SemaphoreType.DMA((N_DEV,)),
                pltpu.SemaphoreType.DMA((N_DEV,)),
            ],
            compiler_params=pltpu.CompilerParams(collective_id=0),
        )(x, w_mat)

    def body(x_ref, w_ref, out_ref, comm_ref, recv_buf, send_sems, recv_sems):
        me = lax.axis_index("i")

        if _KVAR == "wcopy":
            out_ref[pl.ds(0, m_per), :] = w_ref[0:m_per, 0:n_per] + x_ref[:, 0:n_per]
            return

        if _KVAR == "bf16":
            y = jnp.dot(
                x_ref[:, :].astype(jnp.bfloat16),
                w_ref[:, :].astype(jnp.bfloat16),
                preferred_element_type=jnp.float32,
            )
        elif _KVAR == "hi":
            y = jnp.dot(
                x_ref[:, :], w_ref[:, :],
                preferred_element_type=jnp.float32,
                precision=lax.Precision.HIGHEST,
            )
        else:
            y = jnp.dot(x_ref[:, :], w_ref[:, :], preferred_element_type=jnp.float32)
        if _KVAR != "gemm":
            c = 0.7978845608028654
            y = 0.5 * y * (1.0 + jnp.tanh(c * (y + 0.044715 * y * y * y)))

        if _KVAR in ("gemm", "gemmgelu", "bf16", "hi"):
            out_ref[pl.ds(0, m_per), :] = y[:, 0:n_per]
            return

        for j in range(N_DEV):
            comm_ref[j, :, :] = y[:, j * n_per:(j + 1) * n_per]

        if _KVAR == "compute":
            for j in range(N_DEV):
                out_ref[pl.ds(j * m_per, m_per), :] = comm_ref[j, :, :]
            return

        if _KVAR in ("bar", "v2", "r1", "r4", "r1s", "v2s", "r1t", "r1l"):
            barrier_sem = pltpu.get_barrier_semaphore()
            for d in range(1, N_DEV):
                tgt = lax.rem(me + d, N_DEV)
                pl.semaphore_signal(
                    barrier_sem, inc=1,
                    device_id=(tgt,), device_id_type=pl.DeviceIdType.MESH,
                )
            pl.semaphore_wait(barrier_sem, N_DEV - 1)

        n_rdma = {"bar": 0, "r1": 1, "r1s": 1, "r1t": 1, "r1l": 1, "r4": 4}.get(
            _KVAR, N_DEV - 1)
        to_scratch = _KVAR in ("r1s", "v2s", "r1t", "r1l")
        rdmas = []
        for d in range(1, 1 + n_rdma):
            tgt = lax.rem(me + d, N_DEV)
            if _KVAR == "r1t":
                rdma = pltpu.make_async_remote_copy(
                    src_ref=comm_ref.at[tgt, pl.ds(0, 8), :],
                    dst_ref=recv_buf.at[d, pl.ds(0, 8), :],
                    send_sem=send_sems.at[d],
                    recv_sem=recv_sems.at[d],
                    device_id=(tgt,),
                    device_id_type=pl.DeviceIdType.MESH,
                )
            elif _KVAR == "r1l":
                rdma = pltpu.make_async_remote_copy(
                    src_ref=comm_ref.at[tgt],
                    dst_ref=recv_buf.at[d],
                    send_sem=send_sems.at[d],
                    recv_sem=recv_sems.at[d],
                    device_id=tgt,
                    device_id_type=pl.DeviceIdType.LOGICAL,
                )
            else:
                rdma = pltpu.make_async_remote_copy(
                    src_ref=comm_ref.at[tgt],
                    dst_ref=recv_buf.at[d] if to_scratch
                    else out_ref.at[pl.ds(me * m_per, m_per), :],
                    send_sem=send_sems.at[d],
                    recv_sem=recv_sems.at[d],
                    device_id=(tgt,),
                    device_id_type=pl.DeviceIdType.MESH,
                )
            rdma.start()
            rdmas.append(rdma)

        out_ref[pl.ds(me * m_per, m_per), :] = comm_ref[me, :, :]
        if _KVAR in ("bar", "r1", "r4", "r1s", "r1t", "r1l"):
            for j in range(N_DEV):
                out_ref[pl.ds(j * m_per, m_per), :] = comm_ref[j, :, :]

        for rdma in rdmas:
            rdma.wait()

        if to_scratch and _KVAR == "v2s":
            for d in range(1, N_DEV):
                src = lax.rem(me - d + N_DEV, N_DEV)
                out_ref[pl.ds(src * m_per, m_per), :] = recv_buf[d, :, :]

    params = {}
    if _KVAR in ("bar", "v2", "r1", "r4", "r1s", "v2s", "r1t", "r1l"):
        params["compiler_params"] = pltpu.CompilerParams(collective_id=0)
    return pl.pallas_call(
        body,
        out_shape=jax.ShapeDtypeStruct((N_DEV * m_per, n_per), jnp.float32),
        in_specs=[
            pl.BlockSpec(memory_space=pltpu.VMEM),
            pl.BlockSpec(memory_space=pltpu.VMEM),
        ],
        out_specs=pl.BlockSpec(memory_space=pltpu.VMEM),
        scratch_shapes=[
            pltpu.VMEM((N_DEV, m_per, n_per), jnp.float32),
            pltpu.VMEM((N_DEV, m_per, n_per), jnp.float32),
            pltpu.SemaphoreType.DMA((N_DEV,)),
            pltpu.SemaphoreType.DMA((N_DEV,)),
        ],
        **params,
    )(x, w_mat)
